# baseline (speedup 1.0000x reference)
import functools

import jax
import jax.numpy as jnp
from jax import lax
from jax.experimental import pallas as pl
from jax.experimental.pallas import tpu as pltpu
from jax.experimental.pallas import tpu_sc as plsc

SIMILAR_NUM = 16
KNN_K = 20
BLK = 256
NEG = -jnp.inf


def _leaky(x):
    return jnp.where(x >= 0, x, 0.2 * x)


def _argmax_rows(a, lanes, sentinel):
    m = jnp.max(a, axis=1)
    eq = a == m[:, None]
    return jnp.min(jnp.where(eq, lanes, sentinel), axis=1)


_SL = 8
_SLR = 4


def _fps_body(x_ref, y_ref, z_ref, cx_ref, cy_ref, cz_ref, ci_ref):
    B, SL, MC = x_ref.shape
    M = SL * MC
    _, SLR, NC_ = ci_ref.shape
    N = SLR * NC_
    x = x_ref[...]
    y = y_ref[...]
    z = z_ref[...]
    lin = (
        jax.lax.broadcasted_iota(jnp.int32, (B, SL, MC), 1) * MC
        + jax.lax.broadcasted_iota(jnp.int32, (B, SL, MC), 2)
    )
    linr = (
        jax.lax.broadcasted_iota(jnp.int32, (B, SLR, NC_), 1) * NC_
        + jax.lax.broadcasted_iota(jnp.int32, (B, SLR, NC_), 2)
    )

    def _bsum(a):
        return jnp.sum(jnp.sum(a, axis=2), axis=1)

    def _e(v):
        return v[:, None, None]

    def step(t, carry):
        dist, far = carry
        sel = lin == _e(far)
        cx = _bsum(jnp.where(sel, x, 0.0))
        cy = _bsum(jnp.where(sel, y, 0.0))
        cz = _bsum(jnp.where(sel, z, 0.0))
        rec = linr == t
        ci_ref[...] = jnp.where(rec, _e(far), ci_ref[...])
        cx_ref[...] = jnp.where(rec, _e(cx), cx_ref[...])
        cy_ref[...] = jnp.where(rec, _e(cy), cy_ref[...])
        cz_ref[...] = jnp.where(rec, _e(cz), cz_ref[...])
        dx = x - _e(cx)
        dy = y - _e(cy)
        dz = z - _e(cz)
        d = (dx * dx + dy * dy) + dz * dz
        dist = jnp.minimum(dist, d)
        m = jnp.max(jnp.max(dist, axis=2), axis=1)
        far = jnp.min(
            jnp.min(jnp.where(dist == _e(m), lin, M), axis=2), axis=1
        )
        return dist, far

    init = (
        jnp.full((B, SL, MC), 1e10, jnp.float32),
        jnp.zeros((B,), jnp.int32),
    )
    jax.lax.fori_loop(0, N, step, init)


def _fps_call(qa, npoint):
    B, M, _ = qa.shape
    MC = M // _SL
    NC_ = npoint // _SLR
    outs = (
        jax.ShapeDtypeStruct((B, _SLR, NC_), jnp.float32),
        jax.ShapeDtypeStruct((B, _SLR, NC_), jnp.float32),
        jax.ShapeDtypeStruct((B, _SLR, NC_), jnp.float32),
        jax.ShapeDtypeStruct((B, _SLR, NC_), jnp.int32),
    )
    planes = [qa[..., c].reshape(B, _SL, MC) for c in range(3)]
    cx, cy, cz, ci = pl.pallas_call(_fps_body, out_shape=outs)(*planes)
    q_s = jnp.stack(
        [v.reshape(B, npoint) for v in (cx, cy, cz)], axis=-1
    )
    return q_s, ci.reshape(B, npoint)


def _sc_gather(table, idx):
    NC, NS = 2, 16
    NW = NC * NS
    G = idx.shape[0]
    D = table.shape[1]
    CH = 128
    per_w = G // NW
    nch = per_w // CH
    mesh = plsc.VectorSubcoreMesh(core_axis_name="c", subcore_axis_name="s")

    NB_ = max(1, min(4, nch, 65536 // (CH * D)))

    @functools.partial(
        pl.kernel,
        mesh=mesh,
        compiler_params=pltpu.CompilerParams(use_tc_tiling_on_sc=False),
        out_type=jax.ShapeDtypeStruct((G, D), jnp.float32),
        scratch_types=(
            [pltpu.VMEM((CH,), jnp.int32) for _ in range(NB_)]
            + [pltpu.VMEM((CH, D), jnp.float32) for _ in range(NB_)]
            + [pltpu.SemaphoreType.DMA for _ in range(NB_)]
        ),
    )
    def k(idx_hbm, table_hbm, out_hbm, *bufs):
        ibuf = bufs[:NB_]
        rbuf = bufs[NB_ : 2 * NB_]
        sem = bufs[2 * NB_ :]
        wid = lax.axis_index("s") * NC + lax.axis_index("c")
        base = wid * per_w
        cps = [None] * NB_

        def drain(wc):
            cps[wc % NB_].wait()
            pltpu.sync_copy(
                rbuf[wc % NB_], out_hbm.at[pl.ds(base + wc * CH, CH)]
            )

        for c in range(nch):
            s = c % NB_
            pltpu.sync_copy(idx_hbm.at[pl.ds(base + c * CH, CH)], ibuf[s])
            cps[s] = pltpu.async_copy(table_hbm.at[ibuf[s]], rbuf[s], sem[s])
            if c >= NB_ - 1:
                drain(c - (NB_ - 1))
        for wc in range(max(0, nch - NB_ + 1), nch):
            drain(wc)

    return k(idx, table)


def _knn_body(xb_ref, xt_ref, sqb_ref, sqa_ref, si_ref, *, k, n_total):
    xb = xb_ref[0]
    xt = xt_ref[0]
    R, N = xb.shape[0], xt.shape[1]
    sq_all = sqa_ref[0, 0]
    sq_b = sqb_ref[0, 0]
    nd = (
        2.0 * jnp.dot(xb, xt, preferred_element_type=jnp.float32)
        - sq_b[:, None]
        - sq_all[None, :]
    )
    lanes = jax.lax.broadcasted_iota(jnp.int32, (R, N), 1)
    cols = jax.lax.broadcasted_iota(jnp.int32, (R, k), 1)
    si = jnp.zeros((R, k), jnp.int32)
    for kk in range(k):
        idx = _argmax_rows(nd, lanes, N)
        si = jnp.where(cols == kk, idx[:, None], si)
        nd = jnp.where(lanes == idx[:, None], NEG, nd)
    b = pl.program_id(0)
    si_ref[0] = si + b * n_total


def _knn_call(x, k):
    B, N, C = x.shape
    xt = jnp.swapaxes(x, 1, 2)
    sq = jnp.sum(x * x, axis=-1).reshape(B, 1, N)
    nb = N // BLK
    return pl.pallas_call(
        functools.partial(_knn_body, k=k, n_total=N),
        grid=(B, nb),
        in_specs=[
            pl.BlockSpec((1, BLK, C), lambda b_, i: (b_, i, 0)),
            pl.BlockSpec((1, C, N), lambda b_, i: (b_, 0, 0)),
            pl.BlockSpec((1, 1, BLK), lambda b_, i: (b_, 0, i)),
            pl.BlockSpec((1, 1, N), lambda b_, i: (b_, 0, 0)),
        ],
        out_specs=pl.BlockSpec((1, BLK, k), lambda b_, i: (b_, i, 0)),
        out_shape=jax.ShapeDtypeStruct((B, N, k), jnp.int32),
    )(x, xt, sq, sq)


def _pool_body(nbr_ref, xb_ref, w_ref, b_ref, o_ref, *, k, c_in):
    nbr = nbr_ref[0][:, :c_in]
    xb = xb_ref[0]
    R, C = xb.shape
    xrep = jnp.broadcast_to(xb[:, None, :], (R, k, C)).reshape(R * k, C)
    e = jnp.concatenate([nbr - xrep, xrep], axis=1)
    h = jnp.dot(e, w_ref[...], preferred_element_type=jnp.float32) + b_ref[...]
    lk = _leaky(h).reshape(R, k, w_ref.shape[1])
    o_ref[0] = jnp.max(lk, axis=1)


def _pool_call(nbr, x, W, b, k):
    B, N, C = x.shape
    Dp = nbr.shape[1]
    Co = W.shape[1]
    nbr3 = nbr.reshape(B, N * k, Dp)
    return pl.pallas_call(
        functools.partial(_pool_body, k=k, c_in=C),
        grid=(B, N // BLK),
        in_specs=[
            pl.BlockSpec((1, BLK * k, Dp), lambda b_, i: (b_, i, 0)),
            pl.BlockSpec((1, BLK, C), lambda b_, i: (b_, i, 0)),
            pl.BlockSpec((2 * C, Co), lambda b_, i: (0, 0)),
            pl.BlockSpec((1, Co), lambda b_, i: (0, 0)),
        ],
        out_specs=pl.BlockSpec((1, BLK, Co), lambda b_, i: (b_, i, 0)),
        out_shape=jax.ShapeDtypeStruct((B, N, Co), jnp.float32),
    )(nbr3, x, W, b.reshape(1, Co))


def _edge_conv(x, W, b, k):
    B, N, C = x.shape
    idx = _knn_call(x, k)
    if C % 16:
        Dp = 16
        tab = jnp.concatenate(
            [x, jnp.zeros((B, N, Dp - C), jnp.float32)], axis=-1
        ).reshape(B * N, Dp)
    else:
        Dp = C
        tab = x.reshape(B * N, C)
    nbr = _sc_gather(tab, idx.reshape(B * N * k))
    return _pool_call(nbr, x, W, b, k)


def _w4_body(x1_ref, x2_ref, x3_ref, w_ref, b_ref, f_ref):
    cat = jnp.concatenate([x1_ref[0], x2_ref[0], x3_ref[0]], axis=1)
    h = jnp.dot(cat, w_ref[...], preferred_element_type=jnp.float32) + b_ref[...]
    f_ref[0] = _leaky(h)


def _w4_call(x1, x2, x3, W4, b4):
    B, N, _ = x1.shape
    C1, C2, C3 = x1.shape[2], x2.shape[2], x3.shape[2]
    Ct, Co = W4.shape
    f = jnp.float32
    return pl.pallas_call(
        _w4_body,
        grid=(B,),
        in_specs=[
            pl.BlockSpec((1, N, C1), lambda b_: (b_, 0, 0)),
            pl.BlockSpec((1, N, C2), lambda b_: (b_, 0, 0)),
            pl.BlockSpec((1, N, C3), lambda b_: (b_, 0, 0)),
            pl.BlockSpec((Ct, Co), lambda b_: (0, 0)),
            pl.BlockSpec((1, Co), lambda b_: (0, 0)),
        ],
        out_specs=pl.BlockSpec((1, N, Co), lambda b_: (b_, 0, 0)),
        out_shape=jax.ShapeDtypeStruct((B, N, Co), f),
    )(x1, x2, x3, W4, b4.reshape(1, Co))


def _dgcnn(x, W1, b1, W2, b2, W3, b3, W4, b4, k):
    x1 = _edge_conv(x, W1, b1, k)
    x2 = _edge_conv(x1, W2, b2, k)
    x3 = _edge_conv(x2, W3, b3, k)
    ff = _w4_call(x1, x2, x3, W4, b4)
    ffn = ff / (jnp.linalg.norm(ff, axis=-1, keepdims=True) + 1e-8)
    return ff, ffn


def _sim_body(ffqn_ref, ffpnt_ref, qs_ref, pt_ref, sqq_ref, sqp_ref, w_ref,
              si_ref, sig_ref):
    ffqn = ffqn_ref[0]
    ffpnt = ffpnt_ref[0]
    qs = qs_ref[0]
    pt = pt_ref[0]
    R = ffqn.shape[0]
    N = ffpnt.shape[1]
    lanes = jax.lax.broadcasted_iota(jnp.int32, (R, N), 1)

    w1 = jnp.dot(ffqn, ffpnt, preferred_element_type=jnp.float32)
    w = jnp.exp(w1)

    cross = jnp.dot(qs, pt, preferred_element_type=jnp.float32)
    sq_q = sqq_ref[0, 0]
    sq_p = sqp_ref[0, 0]
    d = (sq_q[:, None] - 2.0 * cross) + sq_p[None, :]
    w2 = jnp.exp(-d)
    for _ in range(SIMILAR_NUM // 4):
        v = jnp.max(w2, axis=1)
        idx = jnp.min(jnp.where(w2 == v[:, None], lanes, N), axis=1)
        oh = lanes == idx[:, None]
        w = w + jnp.where(oh, v[:, None], 0.0)
        w2 = jnp.where(oh, NEG, w2)
    w_ref[0] = w

    cols = jax.lax.broadcasted_iota(jnp.int32, (R, SIMILAR_NUM), 1)
    si = jnp.zeros((R, SIMILAR_NUM), jnp.int32)
    wk = w
    for kk in range(SIMILAR_NUM):
        idx = _argmax_rows(wk, lanes, N)
        si = jnp.where(cols == kk, idx[:, None], si)
        wk = jnp.where(lanes == idx[:, None], NEG, wk)
    si_ref[0] = si
    sig_ref[0] = si + pl.program_id(0) * N


def _sim_call(ffqn, ffpn, q_s, p):
    B, N, C = ffqn.shape
    ffpnt = jnp.swapaxes(ffpn, 1, 2)
    pt = jnp.swapaxes(p, 1, 2)
    sqq = jnp.sum(q_s * q_s, axis=-1).reshape(B, 1, N)
    sqp = jnp.sum(p * p, axis=-1).reshape(B, 1, N)
    nb = N // BLK
    f = jnp.float32
    return pl.pallas_call(
        _sim_body,
        grid=(B, nb),
        in_specs=[
            pl.BlockSpec((1, BLK, C), lambda b_, i: (b_, i, 0)),
            pl.BlockSpec((1, C, N), lambda b_, i: (b_, 0, 0)),
            pl.BlockSpec((1, BLK, 3), lambda b_, i: (b_, i, 0)),
            pl.BlockSpec((1, 3, N), lambda b_, i: (b_, 0, 0)),
            pl.BlockSpec((1, 1, BLK), lambda b_, i: (b_, 0, i)),
            pl.BlockSpec((1, 1, N), lambda b_, i: (b_, 0, 0)),
        ],
        out_specs=[
            pl.BlockSpec((1, BLK, N), lambda b_, i: (b_, i, 0)),
            pl.BlockSpec((1, BLK, SIMILAR_NUM), lambda b_, i: (b_, i, 0)),
            pl.BlockSpec((1, BLK, SIMILAR_NUM), lambda b_, i: (b_, i, 0)),
        ],
        out_shape=[
            jax.ShapeDtypeStruct((B, N, N), f),
            jax.ShapeDtypeStruct((B, N, SIMILAR_NUM), jnp.int32),
            jax.ShapeDtypeStruct((B, N, SIMILAR_NUM), jnp.int32),
        ],
    )(ffqn, ffpnt, q_s, pt, sqq, sqp)


def _refine_body(ffq_ref, feat_ref, r1a_ref, r1bc_ref, rb1_ref, r2_ref,
                 rb2_ref, r3_ref, rb3_ref, qs_ref, cf_ref, o_ref, *, thresh):
    f32 = jnp.float32
    R, C = ffq_ref.shape[1], ffq_ref.shape[2]
    feat = feat_ref[0].reshape(R, SIMILAR_NUM, C)
    fmax = jnp.max(feat, axis=1)
    favg = jnp.sum(feat, axis=1) * (1.0 / SIMILAR_NUM)
    f = jnp.concatenate([favg, fmax], axis=1)
    h = (
        jnp.dot(ffq_ref[0], r1a_ref[...], preferred_element_type=f32)
        + jnp.dot(f, r1bc_ref[...], preferred_element_type=f32)
        + rb1_ref[...]
    )
    h = _leaky(h)
    h = _leaky(jnp.dot(h, r2_ref[...], preferred_element_type=f32) + rb2_ref[...])
    v = jnp.dot(h, r3_ref[...], preferred_element_type=f32) + rb3_ref[...]
    mask = (cf_ref[0, 0] >= thresh).astype(f32)
    o_ref[0] = qs_ref[0] + v * mask[:, None]


def _refine_call(ffq, feats, R1, rb1, R2, rb2, R3, rb3, q_s, cf, n_p):
    B, N, C = ffq.shape
    feat3 = feats.reshape(B, N * SIMILAR_NUM, C)
    r1a = R1[:C]
    r1bc = R1[C:]
    C2 = R2.shape[1]
    nb = N // BLK
    f = jnp.float32
    return pl.pallas_call(
        functools.partial(_refine_body, thresh=float(n_p)),
        grid=(B, nb),
        in_specs=[
            pl.BlockSpec((1, BLK, C), lambda b_, i: (b_, i, 0)),
            pl.BlockSpec((1, BLK * SIMILAR_NUM, C), lambda b_, i: (b_, i, 0)),
            pl.BlockSpec((C, C), lambda b_, i: (0, 0)),
            pl.BlockSpec((2 * C, C), lambda b_, i: (0, 0)),
            pl.BlockSpec((1, C), lambda b_, i: (0, 0)),
            pl.BlockSpec((C, C2), lambda b_, i: (0, 0)),
            pl.BlockSpec((1, C2), lambda b_, i: (0, 0)),
            pl.BlockSpec((C2, 3), lambda b_, i: (0, 0)),
            pl.BlockSpec((1, 3), lambda b_, i: (0, 0)),
            pl.BlockSpec((1, BLK, 3), lambda b_, i: (b_, i, 0)),
            pl.BlockSpec((1, 1, BLK), lambda b_, i: (b_, 0, i)),
        ],
        out_specs=pl.BlockSpec((1, BLK, 3), lambda b_, i: (b_, i, 0)),
        out_shape=jax.ShapeDtypeStruct((B, N, 3), f),
    )(ffq, feat3, r1a, r1bc, rb1.reshape(1, C), R2,
      rb2.reshape(1, C2), R3, rb3.reshape(1, 3), q_s, cf)


def kernel(p, q, W1, b1, W2, b2, W3, b3, W4, b4, R1, rb1, R2, rb2, R3, rb3):
    B, N, _ = q.shape
    qa = jnp.concatenate([p, q], axis=1)
    q_s, centroids = _fps_call(qa, N)

    ffp, ffpn = _dgcnn(p, W1, b1, W2, b2, W3, b3, W4, b4, KNN_K)
    ffq, ffqn = _dgcnn(q_s, W1, b1, W2, b2, W3, b3, W4, b4, KNN_K)

    w, sidx, sidxg = _sim_call(ffqn, ffpn, q_s, p)
    feats = _sc_gather(
        ffp.reshape(B * N, ffp.shape[2]), sidxg.reshape(B * N * SIMILAR_NUM)
    )

    cf = centroids.astype(jnp.float32).reshape(B, 1, N)
    q_refine = _refine_call(
        ffq, feats, R1, rb1, R2, rb2, R3, rb3, q_s, cf, p.shape[1]
    )
    return (q_s, q_refine, q_refine, sidx, w)


# speedup vs baseline: 1.1116x; 1.1116x over previous
import functools

import jax
import jax.numpy as jnp
from jax import lax
from jax.experimental import pallas as pl
from jax.experimental.pallas import tpu as pltpu
from jax.experimental.pallas import tpu_sc as plsc

SIMILAR_NUM = 16
KNN_K = 20
BLK = 256
NEG = -jnp.inf


def _leaky(x):
    return jnp.where(x >= 0, x, 0.2 * x)


def _argmax_rows(a, lanes, sentinel):
    m = jnp.max(a, axis=1)
    eq = a == m[:, None]
    return jnp.min(jnp.where(eq, lanes, sentinel), axis=1)


def _fps_body(x_ref, y_ref, z_ref, cx_ref, cy_ref, cz_ref, ci_ref):
    B, M = x_ref.shape
    N = ci_ref.shape[1]
    x = x_ref[...]
    y = y_ref[...]
    z = z_ref[...]
    mcols = jax.lax.broadcasted_iota(jnp.int32, (B, M), 1)
    ncols = jax.lax.broadcasted_iota(jnp.int32, (B, N), 1)

    def step(t, carry):
        dist, far = carry
        sel = mcols == far[:, None]
        cx = jnp.sum(jnp.where(sel, x, 0.0), axis=1)
        cy = jnp.sum(jnp.where(sel, y, 0.0), axis=1)
        cz = jnp.sum(jnp.where(sel, z, 0.0), axis=1)
        rec = ncols == t
        ci_ref[...] = jnp.where(rec, far[:, None], ci_ref[...])
        cx_ref[...] = jnp.where(rec, cx[:, None], cx_ref[...])
        cy_ref[...] = jnp.where(rec, cy[:, None], cy_ref[...])
        cz_ref[...] = jnp.where(rec, cz[:, None], cz_ref[...])
        dx = x - cx[:, None]
        dy = y - cy[:, None]
        dz = z - cz[:, None]
        d = (dx * dx + dy * dy) + dz * dz
        dist = jnp.minimum(dist, d)
        m = jnp.max(dist, axis=1)
        far = jnp.min(jnp.where(dist == m[:, None], mcols, M), axis=1)
        return dist, far

    init = (jnp.full((B, M), 1e10, jnp.float32), jnp.zeros((B,), jnp.int32))
    jax.lax.fori_loop(0, N, step, init)


def _fps_call(qa, npoint):
    B, M, _ = qa.shape
    outs = (
        jax.ShapeDtypeStruct((B, npoint), jnp.float32),
        jax.ShapeDtypeStruct((B, npoint), jnp.float32),
        jax.ShapeDtypeStruct((B, npoint), jnp.float32),
        jax.ShapeDtypeStruct((B, npoint), jnp.int32),
    )
    cx, cy, cz, ci = pl.pallas_call(_fps_body, out_shape=outs)(
        qa[..., 0], qa[..., 1], qa[..., 2]
    )
    return jnp.stack([cx, cy, cz], axis=-1), ci


def _sc_gather(table, idx):
    NC, NS = 2, 16
    NW = NC * NS
    G = idx.shape[0]
    D = table.shape[1]
    CH = 128
    per_w = G // NW
    nch = per_w // CH
    mesh = plsc.VectorSubcoreMesh(core_axis_name="c", subcore_axis_name="s")

    NB_ = max(1, min(4, nch, 65536 // (CH * D)))

    @functools.partial(
        pl.kernel,
        mesh=mesh,
        compiler_params=pltpu.CompilerParams(use_tc_tiling_on_sc=False),
        out_type=jax.ShapeDtypeStruct((G, D), jnp.float32),
        scratch_types=(
            [pltpu.VMEM((CH,), jnp.int32) for _ in range(NB_)]
            + [pltpu.VMEM((CH, D), jnp.float32) for _ in range(NB_)]
            + [pltpu.SemaphoreType.DMA for _ in range(NB_)]
        ),
    )
    def k(idx_hbm, table_hbm, out_hbm, *bufs):
        ibuf = bufs[:NB_]
        rbuf = bufs[NB_ : 2 * NB_]
        sem = bufs[2 * NB_ :]
        wid = lax.axis_index("s") * NC + lax.axis_index("c")
        base = wid * per_w
        cps = [None] * NB_

        def drain(wc):
            cps[wc % NB_].wait()
            pltpu.sync_copy(
                rbuf[wc % NB_], out_hbm.at[pl.ds(base + wc * CH, CH)]
            )

        for c in range(nch):
            s = c % NB_
            pltpu.sync_copy(idx_hbm.at[pl.ds(base + c * CH, CH)], ibuf[s])
            cps[s] = pltpu.async_copy(table_hbm.at[ibuf[s]], rbuf[s], sem[s])
            if c >= NB_ - 1:
                drain(c - (NB_ - 1))
        for wc in range(max(0, nch - NB_ + 1), nch):
            drain(wc)

    return k(idx, table)


def _knn_body(xb_ref, xt_ref, sqb_ref, sqa_ref, si_ref, *, k, n_total):
    xb = xb_ref[0]
    xt = xt_ref[0]
    R, N = xb.shape[0], xt.shape[1]
    sq_all = sqa_ref[0, 0]
    sq_b = sqb_ref[0, 0]
    nd = (
        2.0 * jnp.dot(xb, xt, preferred_element_type=jnp.float32)
        - sq_b[:, None]
        - sq_all[None, :]
    )
    lanes = jax.lax.broadcasted_iota(jnp.int32, (R, N), 1)
    cols = jax.lax.broadcasted_iota(jnp.int32, (R, k), 1)
    si = jnp.zeros((R, k), jnp.int32)
    for kk in range(k):
        idx = _argmax_rows(nd, lanes, N)
        si = jnp.where(cols == kk, idx[:, None], si)
        nd = jnp.where(lanes == idx[:, None], NEG, nd)
    b = pl.program_id(0)
    si_ref[0] = si + b * n_total


def _knn_call(x, k):
    B, N, C = x.shape
    xt = jnp.swapaxes(x, 1, 2)
    sq = jnp.sum(x * x, axis=-1).reshape(B, 1, N)
    nb = N // BLK
    return pl.pallas_call(
        functools.partial(_knn_body, k=k, n_total=N),
        grid=(B, nb),
        in_specs=[
            pl.BlockSpec((1, BLK, C), lambda b_, i: (b_, i, 0)),
            pl.BlockSpec((1, C, N), lambda b_, i: (b_, 0, 0)),
            pl.BlockSpec((1, 1, BLK), lambda b_, i: (b_, 0, i)),
            pl.BlockSpec((1, 1, N), lambda b_, i: (b_, 0, 0)),
        ],
        out_specs=pl.BlockSpec((1, BLK, k), lambda b_, i: (b_, i, 0)),
        out_shape=jax.ShapeDtypeStruct((B, N, k), jnp.int32),
    )(x, xt, sq, sq)


def _pool_body(nbr_ref, xb_ref, w_ref, b_ref, o_ref, *, k, c_in):
    nbr = nbr_ref[0][:, :c_in]
    xb = xb_ref[0]
    R, C = xb.shape
    xrep = jnp.broadcast_to(xb[:, None, :], (R, k, C)).reshape(R * k, C)
    e = jnp.concatenate([nbr - xrep, xrep], axis=1)
    h = jnp.dot(e, w_ref[...], preferred_element_type=jnp.float32) + b_ref[...]
    lk = _leaky(h).reshape(R, k, w_ref.shape[1])
    o_ref[0] = jnp.max(lk, axis=1)


def _pool_call(nbr, x, W, b, k):
    B, N, C = x.shape
    Dp = nbr.shape[1]
    Co = W.shape[1]
    nbr3 = nbr.reshape(B, N * k, Dp)
    return pl.pallas_call(
        functools.partial(_pool_body, k=k, c_in=C),
        grid=(B, N // BLK),
        in_specs=[
            pl.BlockSpec((1, BLK * k, Dp), lambda b_, i: (b_, i, 0)),
            pl.BlockSpec((1, BLK, C), lambda b_, i: (b_, i, 0)),
            pl.BlockSpec((2 * C, Co), lambda b_, i: (0, 0)),
            pl.BlockSpec((1, Co), lambda b_, i: (0, 0)),
        ],
        out_specs=pl.BlockSpec((1, BLK, Co), lambda b_, i: (b_, i, 0)),
        out_shape=jax.ShapeDtypeStruct((B, N, Co), jnp.float32),
    )(nbr3, x, W, b.reshape(1, Co))


def _edge_conv(x, W, b, k):
    B, N, C = x.shape
    idx = _knn_call(x, k)
    if C % 16:
        Dp = 16
        tab = jnp.concatenate(
            [x, jnp.zeros((B, N, Dp - C), jnp.float32)], axis=-1
        ).reshape(B * N, Dp)
    else:
        Dp = C
        tab = x.reshape(B * N, C)
    nbr = _sc_gather(tab, idx.reshape(B * N * k))
    return _pool_call(nbr, x, W, b, k)


def _w4_body(x1_ref, x2_ref, x3_ref, w_ref, b_ref, f_ref):
    cat = jnp.concatenate([x1_ref[0], x2_ref[0], x3_ref[0]], axis=1)
    h = jnp.dot(cat, w_ref[...], preferred_element_type=jnp.float32) + b_ref[...]
    f_ref[0] = _leaky(h)


def _w4_call(x1, x2, x3, W4, b4):
    B, N, _ = x1.shape
    C1, C2, C3 = x1.shape[2], x2.shape[2], x3.shape[2]
    Ct, Co = W4.shape
    f = jnp.float32
    return pl.pallas_call(
        _w4_body,
        grid=(B,),
        in_specs=[
            pl.BlockSpec((1, N, C1), lambda b_: (b_, 0, 0)),
            pl.BlockSpec((1, N, C2), lambda b_: (b_, 0, 0)),
            pl.BlockSpec((1, N, C3), lambda b_: (b_, 0, 0)),
            pl.BlockSpec((Ct, Co), lambda b_: (0, 0)),
            pl.BlockSpec((1, Co), lambda b_: (0, 0)),
        ],
        out_specs=pl.BlockSpec((1, N, Co), lambda b_: (b_, 0, 0)),
        out_shape=jax.ShapeDtypeStruct((B, N, Co), f),
    )(x1, x2, x3, W4, b4.reshape(1, Co))


def _dgcnn(x, W1, b1, W2, b2, W3, b3, W4, b4, k):
    x1 = _edge_conv(x, W1, b1, k)
    x2 = _edge_conv(x1, W2, b2, k)
    x3 = _edge_conv(x2, W3, b3, k)
    ff = _w4_call(x1, x2, x3, W4, b4)
    ffn = ff / (jnp.linalg.norm(ff, axis=-1, keepdims=True) + 1e-8)
    return ff, ffn


def _sim_body(ffqn_ref, ffpnt_ref, qs_ref, pt_ref, sqq_ref, sqp_ref, w_ref,
              si_ref, sig_ref):
    ffqn = ffqn_ref[0]
    ffpnt = ffpnt_ref[0]
    qs = qs_ref[0]
    pt = pt_ref[0]
    R = ffqn.shape[0]
    N = ffpnt.shape[1]
    lanes = jax.lax.broadcasted_iota(jnp.int32, (R, N), 1)

    w1 = jnp.dot(ffqn, ffpnt, preferred_element_type=jnp.float32)
    w = jnp.exp(w1)

    cross = jnp.dot(qs, pt, preferred_element_type=jnp.float32)
    sq_q = sqq_ref[0, 0]
    sq_p = sqp_ref[0, 0]
    d = (sq_q[:, None] - 2.0 * cross) + sq_p[None, :]
    w2 = jnp.exp(-d)
    for _ in range(SIMILAR_NUM // 4):
        v = jnp.max(w2, axis=1)
        idx = jnp.min(jnp.where(w2 == v[:, None], lanes, N), axis=1)
        oh = lanes == idx[:, None]
        w = w + jnp.where(oh, v[:, None], 0.0)
        w2 = jnp.where(oh, NEG, w2)
    w_ref[0] = w

    cols = jax.lax.broadcasted_iota(jnp.int32, (R, SIMILAR_NUM), 1)
    si = jnp.zeros((R, SIMILAR_NUM), jnp.int32)
    wk = w
    for kk in range(SIMILAR_NUM):
        idx = _argmax_rows(wk, lanes, N)
        si = jnp.where(cols == kk, idx[:, None], si)
        wk = jnp.where(lanes == idx[:, None], NEG, wk)
    si_ref[0] = si
    sig_ref[0] = si + pl.program_id(0) * N


def _sim_call(ffqn, ffpn, q_s, p):
    B, N, C = ffqn.shape
    ffpnt = jnp.swapaxes(ffpn, 1, 2)
    pt = jnp.swapaxes(p, 1, 2)
    sqq = jnp.sum(q_s * q_s, axis=-1).reshape(B, 1, N)
    sqp = jnp.sum(p * p, axis=-1).reshape(B, 1, N)
    nb = N // BLK
    f = jnp.float32
    return pl.pallas_call(
        _sim_body,
        grid=(B, nb),
        in_specs=[
            pl.BlockSpec((1, BLK, C), lambda b_, i: (b_, i, 0)),
            pl.BlockSpec((1, C, N), lambda b_, i: (b_, 0, 0)),
            pl.BlockSpec((1, BLK, 3), lambda b_, i: (b_, i, 0)),
            pl.BlockSpec((1, 3, N), lambda b_, i: (b_, 0, 0)),
            pl.BlockSpec((1, 1, BLK), lambda b_, i: (b_, 0, i)),
            pl.BlockSpec((1, 1, N), lambda b_, i: (b_, 0, 0)),
        ],
        out_specs=[
            pl.BlockSpec((1, BLK, N), lambda b_, i: (b_, i, 0)),
            pl.BlockSpec((1, BLK, SIMILAR_NUM), lambda b_, i: (b_, i, 0)),
            pl.BlockSpec((1, BLK, SIMILAR_NUM), lambda b_, i: (b_, i, 0)),
        ],
        out_shape=[
            jax.ShapeDtypeStruct((B, N, N), f),
            jax.ShapeDtypeStruct((B, N, SIMILAR_NUM), jnp.int32),
            jax.ShapeDtypeStruct((B, N, SIMILAR_NUM), jnp.int32),
        ],
    )(ffqn, ffpnt, q_s, pt, sqq, sqp)


def _refine_body(ffq_ref, feat_ref, r1a_ref, r1bc_ref, rb1_ref, r2_ref,
                 rb2_ref, r3_ref, rb3_ref, qs_ref, cf_ref, o_ref, *, thresh):
    f32 = jnp.float32
    R, C = ffq_ref.shape[1], ffq_ref.shape[2]
    feat = feat_ref[0].reshape(R, SIMILAR_NUM, C)
    fmax = jnp.max(feat, axis=1)
    favg = jnp.sum(feat, axis=1) * (1.0 / SIMILAR_NUM)
    f = jnp.concatenate([favg, fmax], axis=1)
    h = (
        jnp.dot(ffq_ref[0], r1a_ref[...], preferred_element_type=f32)
        + jnp.dot(f, r1bc_ref[...], preferred_element_type=f32)
        + rb1_ref[...]
    )
    h = _leaky(h)
    h = _leaky(jnp.dot(h, r2_ref[...], preferred_element_type=f32) + rb2_ref[...])
    v = jnp.dot(h, r3_ref[...], preferred_element_type=f32) + rb3_ref[...]
    mask = (cf_ref[0, 0] >= thresh).astype(f32)
    o_ref[0] = qs_ref[0] + v * mask[:, None]


def _refine_call(ffq, feats, R1, rb1, R2, rb2, R3, rb3, q_s, cf, n_p):
    B, N, C = ffq.shape
    feat3 = feats.reshape(B, N * SIMILAR_NUM, C)
    r1a = R1[:C]
    r1bc = R1[C:]
    C2 = R2.shape[1]
    nb = N // BLK
    f = jnp.float32
    return pl.pallas_call(
        functools.partial(_refine_body, thresh=float(n_p)),
        grid=(B, nb),
        in_specs=[
            pl.BlockSpec((1, BLK, C), lambda b_, i: (b_, i, 0)),
            pl.BlockSpec((1, BLK * SIMILAR_NUM, C), lambda b_, i: (b_, i, 0)),
            pl.BlockSpec((C, C), lambda b_, i: (0, 0)),
            pl.BlockSpec((2 * C, C), lambda b_, i: (0, 0)),
            pl.BlockSpec((1, C), lambda b_, i: (0, 0)),
            pl.BlockSpec((C, C2), lambda b_, i: (0, 0)),
            pl.BlockSpec((1, C2), lambda b_, i: (0, 0)),
            pl.BlockSpec((C2, 3), lambda b_, i: (0, 0)),
            pl.BlockSpec((1, 3), lambda b_, i: (0, 0)),
            pl.BlockSpec((1, BLK, 3), lambda b_, i: (b_, i, 0)),
            pl.BlockSpec((1, 1, BLK), lambda b_, i: (b_, 0, i)),
        ],
        out_specs=pl.BlockSpec((1, BLK, 3), lambda b_, i: (b_, i, 0)),
        out_shape=jax.ShapeDtypeStruct((B, N, 3), f),
    )(ffq, feat3, r1a, r1bc, rb1.reshape(1, C), R2,
      rb2.reshape(1, C2), R3, rb3.reshape(1, 3), q_s, cf)


def kernel(p, q, W1, b1, W2, b2, W3, b3, W4, b4, R1, rb1, R2, rb2, R3, rb3):
    B, N, _ = q.shape
    qa = jnp.concatenate([p, q], axis=1)
    q_s, centroids = _fps_call(qa, N)

    ffp, ffpn = _dgcnn(p, W1, b1, W2, b2, W3, b3, W4, b4, KNN_K)
    ffq, ffqn = _dgcnn(q_s, W1, b1, W2, b2, W3, b3, W4, b4, KNN_K)

    w, sidx, sidxg = _sim_call(ffqn, ffpn, q_s, p)
    feats = _sc_gather(
        ffp.reshape(B * N, ffp.shape[2]), sidxg.reshape(B * N * SIMILAR_NUM)
    )

    cf = centroids.astype(jnp.float32).reshape(B, 1, N)
    q_refine = _refine_call(
        ffq, feats, R1, rb1, R2, rb2, R3, rb3, q_s, cf, p.shape[1]
    )
    return (q_s, q_refine, q_refine, sidx, w)


# speedup vs baseline: 1.2007x; 1.0802x over previous
import functools

import jax
import jax.numpy as jnp
from jax import lax
from jax.experimental import pallas as pl
from jax.experimental.pallas import tpu as pltpu
from jax.experimental.pallas import tpu_sc as plsc

SIMILAR_NUM = 16
KNN_K = 20
BLK = 512
NEG = -jnp.inf


def _leaky(x):
    return jnp.where(x >= 0, x, 0.2 * x)


def _argmax_rows(a, lanes, sentinel):
    m = jnp.max(a, axis=1)
    eq = a == m[:, None]
    return jnp.min(jnp.where(eq, lanes, sentinel), axis=1)


def _fps_body(x_ref, y_ref, z_ref, cx_ref, cy_ref, cz_ref, ci_ref):
    B, M = x_ref.shape
    N = ci_ref.shape[1]
    x = x_ref[...]
    y = y_ref[...]
    z = z_ref[...]
    mcols = jax.lax.broadcasted_iota(jnp.int32, (B, M), 1)
    ncols = jax.lax.broadcasted_iota(jnp.int32, (B, N), 1)

    def step(t, carry):
        dist, far = carry
        sel = mcols == far[:, None]
        cx = jnp.sum(jnp.where(sel, x, 0.0), axis=1)
        cy = jnp.sum(jnp.where(sel, y, 0.0), axis=1)
        cz = jnp.sum(jnp.where(sel, z, 0.0), axis=1)
        rec = ncols == t
        ci_ref[...] = jnp.where(rec, far[:, None], ci_ref[...])
        cx_ref[...] = jnp.where(rec, cx[:, None], cx_ref[...])
        cy_ref[...] = jnp.where(rec, cy[:, None], cy_ref[...])
        cz_ref[...] = jnp.where(rec, cz[:, None], cz_ref[...])
        dx = x - cx[:, None]
        dy = y - cy[:, None]
        dz = z - cz[:, None]
        d = (dx * dx + dy * dy) + dz * dz
        dist = jnp.minimum(dist, d)
        m = jnp.max(dist, axis=1)
        far = jnp.min(jnp.where(dist == m[:, None], mcols, M), axis=1)
        return dist, far

    init = (jnp.full((B, M), 1e10, jnp.float32), jnp.zeros((B,), jnp.int32))
    jax.lax.fori_loop(0, N, step, init)


def _fps_call(qa, npoint):
    B, M, _ = qa.shape
    outs = (
        jax.ShapeDtypeStruct((B, npoint), jnp.float32),
        jax.ShapeDtypeStruct((B, npoint), jnp.float32),
        jax.ShapeDtypeStruct((B, npoint), jnp.float32),
        jax.ShapeDtypeStruct((B, npoint), jnp.int32),
    )
    cx, cy, cz, ci = pl.pallas_call(_fps_body, out_shape=outs)(
        qa[..., 0], qa[..., 1], qa[..., 2]
    )
    return jnp.stack([cx, cy, cz], axis=-1), ci


def _sc_gather(table, idx):
    NC, NS = 2, 16
    NW = NC * NS
    G = idx.shape[0]
    D = table.shape[1]
    CH = 128
    per_w = G // NW
    nch = per_w // CH
    mesh = plsc.VectorSubcoreMesh(core_axis_name="c", subcore_axis_name="s")

    NB_ = max(1, min(4, nch, 65536 // (CH * D)))

    @functools.partial(
        pl.kernel,
        mesh=mesh,
        compiler_params=pltpu.CompilerParams(use_tc_tiling_on_sc=False),
        out_type=jax.ShapeDtypeStruct((G, D), jnp.float32),
        scratch_types=(
            [pltpu.VMEM((CH,), jnp.int32) for _ in range(NB_)]
            + [pltpu.VMEM((CH, D), jnp.float32) for _ in range(NB_)]
            + [pltpu.SemaphoreType.DMA for _ in range(NB_)]
        ),
    )
    def k(idx_hbm, table_hbm, out_hbm, *bufs):
        ibuf = bufs[:NB_]
        rbuf = bufs[NB_ : 2 * NB_]
        sem = bufs[2 * NB_ :]
        wid = lax.axis_index("s") * NC + lax.axis_index("c")
        base = wid * per_w
        cps = [None] * NB_

        def drain(wc):
            cps[wc % NB_].wait()
            pltpu.sync_copy(
                rbuf[wc % NB_], out_hbm.at[pl.ds(base + wc * CH, CH)]
            )

        for c in range(nch):
            s = c % NB_
            pltpu.sync_copy(idx_hbm.at[pl.ds(base + c * CH, CH)], ibuf[s])
            cps[s] = pltpu.async_copy(table_hbm.at[ibuf[s]], rbuf[s], sem[s])
            if c >= NB_ - 1:
                drain(c - (NB_ - 1))
        for wc in range(max(0, nch - NB_ + 1), nch):
            drain(wc)

    return k(idx, table)


def _knn_body(xb_ref, xt_ref, sqb_ref, sqa_ref, si_ref, *, k, n_total):
    xb = xb_ref[0]
    xt = xt_ref[0]
    R, N = xb.shape[0], xt.shape[1]
    sq_all = sqa_ref[0, 0]
    sq_b = sqb_ref[0, 0]
    nd = (
        2.0 * jnp.dot(xb, xt, preferred_element_type=jnp.float32)
        - sq_b[:, None]
        - sq_all[None, :]
    )
    lanes = jax.lax.broadcasted_iota(jnp.int32, (R, N), 1)
    cols = jax.lax.broadcasted_iota(jnp.int32, (R, k), 1)
    si = jnp.zeros((R, k), jnp.int32)
    for kk in range(k):
        idx = _argmax_rows(nd, lanes, N)
        si = jnp.where(cols == kk, idx[:, None], si)
        nd = jnp.where(lanes == idx[:, None], NEG, nd)
    b = pl.program_id(0)
    si_ref[0] = si + b * n_total


def _knn_call(x, k):
    B, N, C = x.shape
    xt = jnp.swapaxes(x, 1, 2)
    sq = jnp.sum(x * x, axis=-1).reshape(B, 1, N)
    nb = N // BLK
    return pl.pallas_call(
        functools.partial(_knn_body, k=k, n_total=N),
        grid=(B, nb),
        in_specs=[
            pl.BlockSpec((1, BLK, C), lambda b_, i: (b_, i, 0)),
            pl.BlockSpec((1, C, N), lambda b_, i: (b_, 0, 0)),
            pl.BlockSpec((1, 1, BLK), lambda b_, i: (b_, 0, i)),
            pl.BlockSpec((1, 1, N), lambda b_, i: (b_, 0, 0)),
        ],
        out_specs=pl.BlockSpec((1, BLK, k), lambda b_, i: (b_, i, 0)),
        out_shape=jax.ShapeDtypeStruct((B, N, k), jnp.int32),
    )(x, xt, sq, sq)


def _pool_body(nbr_ref, xb_ref, w_ref, b_ref, o_ref, *, k, c_in):
    nbr = nbr_ref[0][:, :c_in]
    xb = xb_ref[0]
    R, C = xb.shape
    xrep = jnp.broadcast_to(xb[:, None, :], (R, k, C)).reshape(R * k, C)
    e = jnp.concatenate([nbr - xrep, xrep], axis=1)
    h = jnp.dot(e, w_ref[...], preferred_element_type=jnp.float32) + b_ref[...]
    lk = _leaky(h).reshape(R, k, w_ref.shape[1])
    o_ref[0] = jnp.max(lk, axis=1)


def _pool_call(nbr, x, W, b, k):
    B, N, C = x.shape
    Dp = nbr.shape[1]
    Co = W.shape[1]
    nbr3 = nbr.reshape(B, N * k, Dp)
    return pl.pallas_call(
        functools.partial(_pool_body, k=k, c_in=C),
        grid=(B, N // BLK),
        in_specs=[
            pl.BlockSpec((1, BLK * k, Dp), lambda b_, i: (b_, i, 0)),
            pl.BlockSpec((1, BLK, C), lambda b_, i: (b_, i, 0)),
            pl.BlockSpec((2 * C, Co), lambda b_, i: (0, 0)),
            pl.BlockSpec((1, Co), lambda b_, i: (0, 0)),
        ],
        out_specs=pl.BlockSpec((1, BLK, Co), lambda b_, i: (b_, i, 0)),
        out_shape=jax.ShapeDtypeStruct((B, N, Co), jnp.float32),
    )(nbr3, x, W, b.reshape(1, Co))


def _edge_conv(x, W, b, k):
    B, N, C = x.shape
    idx = _knn_call(x, k)
    if C % 16:
        Dp = 16
        tab = jnp.concatenate(
            [x, jnp.zeros((B, N, Dp - C), jnp.float32)], axis=-1
        ).reshape(B * N, Dp)
    else:
        Dp = C
        tab = x.reshape(B * N, C)
    nbr = _sc_gather(tab, idx.reshape(B * N * k))
    return _pool_call(nbr, x, W, b, k)


def _w4_body(x1_ref, x2_ref, x3_ref, w_ref, b_ref, f_ref):
    cat = jnp.concatenate([x1_ref[0], x2_ref[0], x3_ref[0]], axis=1)
    h = jnp.dot(cat, w_ref[...], preferred_element_type=jnp.float32) + b_ref[...]
    f_ref[0] = _leaky(h)


def _w4_call(x1, x2, x3, W4, b4):
    B, N, _ = x1.shape
    C1, C2, C3 = x1.shape[2], x2.shape[2], x3.shape[2]
    Ct, Co = W4.shape
    f = jnp.float32
    return pl.pallas_call(
        _w4_body,
        grid=(B,),
        in_specs=[
            pl.BlockSpec((1, N, C1), lambda b_: (b_, 0, 0)),
            pl.BlockSpec((1, N, C2), lambda b_: (b_, 0, 0)),
            pl.BlockSpec((1, N, C3), lambda b_: (b_, 0, 0)),
            pl.BlockSpec((Ct, Co), lambda b_: (0, 0)),
            pl.BlockSpec((1, Co), lambda b_: (0, 0)),
        ],
        out_specs=pl.BlockSpec((1, N, Co), lambda b_: (b_, 0, 0)),
        out_shape=jax.ShapeDtypeStruct((B, N, Co), f),
    )(x1, x2, x3, W4, b4.reshape(1, Co))


def _dgcnn(x, W1, b1, W2, b2, W3, b3, W4, b4, k):
    x1 = _edge_conv(x, W1, b1, k)
    x2 = _edge_conv(x1, W2, b2, k)
    x3 = _edge_conv(x2, W3, b3, k)
    ff = _w4_call(x1, x2, x3, W4, b4)
    ffn = ff / (jnp.linalg.norm(ff, axis=-1, keepdims=True) + 1e-8)
    return ff, ffn


def _sim_body(ffqn_ref, ffpnt_ref, qs_ref, pt_ref, sqq_ref, sqp_ref, w_ref,
              si_ref, sig_ref):
    ffqn = ffqn_ref[0]
    ffpnt = ffpnt_ref[0]
    qs = qs_ref[0]
    pt = pt_ref[0]
    R = ffqn.shape[0]
    N = ffpnt.shape[1]
    lanes = jax.lax.broadcasted_iota(jnp.int32, (R, N), 1)

    w1 = jnp.dot(ffqn, ffpnt, preferred_element_type=jnp.float32)
    w = jnp.exp(w1)

    cross = jnp.dot(qs, pt, preferred_element_type=jnp.float32)
    sq_q = sqq_ref[0, 0]
    sq_p = sqp_ref[0, 0]
    d = (sq_q[:, None] - 2.0 * cross) + sq_p[None, :]
    w2 = jnp.exp(-d)
    for _ in range(SIMILAR_NUM // 4):
        v = jnp.max(w2, axis=1)
        idx = jnp.min(jnp.where(w2 == v[:, None], lanes, N), axis=1)
        oh = lanes == idx[:, None]
        w = w + jnp.where(oh, v[:, None], 0.0)
        w2 = jnp.where(oh, NEG, w2)
    w_ref[0] = w

    cols = jax.lax.broadcasted_iota(jnp.int32, (R, SIMILAR_NUM), 1)
    si = jnp.zeros((R, SIMILAR_NUM), jnp.int32)
    wk = w
    for kk in range(SIMILAR_NUM):
        idx = _argmax_rows(wk, lanes, N)
        si = jnp.where(cols == kk, idx[:, None], si)
        wk = jnp.where(lanes == idx[:, None], NEG, wk)
    si_ref[0] = si
    sig_ref[0] = si + pl.program_id(0) * N


def _sim_call(ffqn, ffpn, q_s, p):
    B, N, C = ffqn.shape
    ffpnt = jnp.swapaxes(ffpn, 1, 2)
    pt = jnp.swapaxes(p, 1, 2)
    sqq = jnp.sum(q_s * q_s, axis=-1).reshape(B, 1, N)
    sqp = jnp.sum(p * p, axis=-1).reshape(B, 1, N)
    nb = N // BLK
    f = jnp.float32
    return pl.pallas_call(
        _sim_body,
        grid=(B, nb),
        in_specs=[
            pl.BlockSpec((1, BLK, C), lambda b_, i: (b_, i, 0)),
            pl.BlockSpec((1, C, N), lambda b_, i: (b_, 0, 0)),
            pl.BlockSpec((1, BLK, 3), lambda b_, i: (b_, i, 0)),
            pl.BlockSpec((1, 3, N), lambda b_, i: (b_, 0, 0)),
            pl.BlockSpec((1, 1, BLK), lambda b_, i: (b_, 0, i)),
            pl.BlockSpec((1, 1, N), lambda b_, i: (b_, 0, 0)),
        ],
        out_specs=[
            pl.BlockSpec((1, BLK, N), lambda b_, i: (b_, i, 0)),
            pl.BlockSpec((1, BLK, SIMILAR_NUM), lambda b_, i: (b_, i, 0)),
            pl.BlockSpec((1, BLK, SIMILAR_NUM), lambda b_, i: (b_, i, 0)),
        ],
        out_shape=[
            jax.ShapeDtypeStruct((B, N, N), f),
            jax.ShapeDtypeStruct((B, N, SIMILAR_NUM), jnp.int32),
            jax.ShapeDtypeStruct((B, N, SIMILAR_NUM), jnp.int32),
        ],
    )(ffqn, ffpnt, q_s, pt, sqq, sqp)


def _refine_body(ffq_ref, feat_ref, r1a_ref, r1bc_ref, rb1_ref, r2_ref,
                 rb2_ref, r3_ref, rb3_ref, qs_ref, cf_ref, o_ref, *, thresh):
    f32 = jnp.float32
    R, C = ffq_ref.shape[1], ffq_ref.shape[2]
    feat = feat_ref[0].reshape(R, SIMILAR_NUM, C)
    fmax = jnp.max(feat, axis=1)
    favg = jnp.sum(feat, axis=1) * (1.0 / SIMILAR_NUM)
    f = jnp.concatenate([favg, fmax], axis=1)
    h = (
        jnp.dot(ffq_ref[0], r1a_ref[...], preferred_element_type=f32)
        + jnp.dot(f, r1bc_ref[...], preferred_element_type=f32)
        + rb1_ref[...]
    )
    h = _leaky(h)
    h = _leaky(jnp.dot(h, r2_ref[...], preferred_element_type=f32) + rb2_ref[...])
    v = jnp.dot(h, r3_ref[...], preferred_element_type=f32) + rb3_ref[...]
    mask = (cf_ref[0, 0] >= thresh).astype(f32)
    o_ref[0] = qs_ref[0] + v * mask[:, None]


def _refine_call(ffq, feats, R1, rb1, R2, rb2, R3, rb3, q_s, cf, n_p):
    B, N, C = ffq.shape
    feat3 = feats.reshape(B, N * SIMILAR_NUM, C)
    r1a = R1[:C]
    r1bc = R1[C:]
    C2 = R2.shape[1]
    nb = N // BLK
    f = jnp.float32
    return pl.pallas_call(
        functools.partial(_refine_body, thresh=float(n_p)),
        grid=(B, nb),
        in_specs=[
            pl.BlockSpec((1, BLK, C), lambda b_, i: (b_, i, 0)),
            pl.BlockSpec((1, BLK * SIMILAR_NUM, C), lambda b_, i: (b_, i, 0)),
            pl.BlockSpec((C, C), lambda b_, i: (0, 0)),
            pl.BlockSpec((2 * C, C), lambda b_, i: (0, 0)),
            pl.BlockSpec((1, C), lambda b_, i: (0, 0)),
            pl.BlockSpec((C, C2), lambda b_, i: (0, 0)),
            pl.BlockSpec((1, C2), lambda b_, i: (0, 0)),
            pl.BlockSpec((C2, 3), lambda b_, i: (0, 0)),
            pl.BlockSpec((1, 3), lambda b_, i: (0, 0)),
            pl.BlockSpec((1, BLK, 3), lambda b_, i: (b_, i, 0)),
            pl.BlockSpec((1, 1, BLK), lambda b_, i: (b_, 0, i)),
        ],
        out_specs=pl.BlockSpec((1, BLK, 3), lambda b_, i: (b_, i, 0)),
        out_shape=jax.ShapeDtypeStruct((B, N, 3), f),
    )(ffq, feat3, r1a, r1bc, rb1.reshape(1, C), R2,
      rb2.reshape(1, C2), R3, rb3.reshape(1, 3), q_s, cf)


def kernel(p, q, W1, b1, W2, b2, W3, b3, W4, b4, R1, rb1, R2, rb2, R3, rb3):
    B, N, _ = q.shape
    qa = jnp.concatenate([p, q], axis=1)
    q_s, centroids = _fps_call(qa, N)

    ffp, ffpn = _dgcnn(p, W1, b1, W2, b2, W3, b3, W4, b4, KNN_K)
    ffq, ffqn = _dgcnn(q_s, W1, b1, W2, b2, W3, b3, W4, b4, KNN_K)

    w, sidx, sidxg = _sim_call(ffqn, ffpn, q_s, p)
    feats = _sc_gather(
        ffp.reshape(B * N, ffp.shape[2]), sidxg.reshape(B * N * SIMILAR_NUM)
    )

    cf = centroids.astype(jnp.float32).reshape(B, 1, N)
    q_refine = _refine_call(
        ffq, feats, R1, rb1, R2, rb2, R3, rb3, q_s, cf, p.shape[1]
    )
    return (q_s, q_refine, q_refine, sidx, w)
